# Initial kernel scaffold; baseline (speedup 1.0000x reference)
import jax
import jax.numpy as jnp
from jax.experimental import pallas as pl


def kernel(node_features, edge_features, latent_features, edge_index, W_node, W_edge, W_msg, W_upd, W_dec1, W_dec2):
    raise NotImplementedError("write your pallas kernel here")


# speedup vs baseline: 2.3568x; 2.3568x over previous
import functools

import jax
import jax.numpy as jnp
from jax import lax
from jax.experimental import pallas as pl
from jax.experimental.pallas import tpu as pltpu
from jax.experimental.pallas import tpu_sc as plsc

N = 10000
E = 320000
L = 128

RB = 1000
NSTEPS = N // RB

NC = 2
NS = 16
NW = NC * NS
EPW = E // NW
C = 80
NCH = EPW // C
NP = 10240
RPT = NP // NS


def _enc_body(nfb_ref, lat_ref, wn0_ref, wn1_ref, we_ref, wm1_ref, wm2_ref,
              wm3_ref, ne_ref, a_ref, b_ref, vpn_ref):
    ne = jnp.maximum(
        nfb_ref[...] * wn0_ref[...]
        + jnp.dot(lat_ref[...], wn1_ref[...], preferred_element_type=jnp.float32),
        0.0)
    ne_ref[...] = ne
    a_ref[...] = jnp.dot(ne, wm1_ref[...], preferred_element_type=jnp.float32)
    b_ref[...] = jnp.dot(ne, wm2_ref[...], preferred_element_type=jnp.float32)
    ep = jnp.maximum(we_ref[...], 0.0)
    en = jnp.maximum(-we_ref[...], 0.0)
    vp = jnp.dot(ep, wm3_ref[...], preferred_element_type=jnp.float32)
    vn = jnp.dot(en, wm3_ref[...], preferred_element_type=jnp.float32)
    vpn_ref[...] = jnp.concatenate([vp, vn], axis=0)


def _encode(nf_b, lat, wn0, wn1, we, wm1, wm2, wm3):
    row = pl.BlockSpec((RB, L), lambda i: (i, 0))
    w1 = pl.BlockSpec((1, L), lambda i: (0, 0))
    wL = pl.BlockSpec((L, L), lambda i: (0, 0))
    return pl.pallas_call(
        _enc_body,
        grid=(NSTEPS,),
        in_specs=[row, row, w1, wL, w1, wL, wL, wL],
        out_specs=[row, row, row, pl.BlockSpec((2, L), lambda i: (0, 0))],
        out_shape=[
            jax.ShapeDtypeStruct((N, L), jnp.float32),
            jax.ShapeDtypeStruct((N, L), jnp.float32),
            jax.ShapeDtypeStruct((N, L), jnp.float32),
            jax.ShapeDtypeStruct((2, L), jnp.float32),
        ],
    )(nf_b, lat, wn0, wn1, we, wm1, wm2, wm3)


def _edge_body(a_hbm, b_hbm, src_hbm, dst_hbm, ef_hbm, vpn_hbm, zer_hbm,
               out0_hbm, out1_hbm,
               agg_sp, sidx_v, didx_v, ef_v, arows_v, brows_v, vpn_v,
               sem_a, sem_b, sem_z):
    cid = lax.axis_index("c")
    sid = lax.axis_index("s")
    wid = cid * NS + sid

    pltpu.async_copy(zer_hbm, agg_sp.at[pl.ds(sid * RPT, RPT)], sem_z).wait()
    pltpu.sync_copy(vpn_hbm, vpn_v)
    plsc.subcore_barrier()

    def chunk(i, carry):
        base = wid * EPW + i * C
        pltpu.sync_copy(src_hbm.at[pl.ds(base, C)], sidx_v)
        pltpu.sync_copy(dst_hbm.at[pl.ds(base, C)], didx_v)
        pltpu.sync_copy(ef_hbm.at[pl.ds(base, C)], ef_v)
        ca = pltpu.async_copy(a_hbm.at[sidx_v], arows_v, sem_a)
        cb = pltpu.async_copy(b_hbm.at[didx_v], brows_v, sem_b)
        ca.wait()
        cb.wait()

        def edge16(q, c2):
            ev = ef_v[pl.ds(q * 16, 16)]
            for rr in range(16):
                s = ev[rr]
                sp = jnp.maximum(s, 0.0)
                sn = jnp.maximum(-s, 0.0)
                r = q * 16 + rr
                for j in range(L // 16):
                    sl = pl.ds(j * 16, 16)
                    v = (arows_v[r, sl] + brows_v[r, sl]
                         + sp * vpn_v[0, sl] + sn * vpn_v[1, sl])
                    arows_v[r, sl] = jnp.maximum(v, 0.0)
            return c2

        lax.fori_loop(0, C // 16, edge16, 0)
        pltpu.sync_copy(arows_v, agg_sp.at[didx_v], add=True)
        return carry

    lax.fori_loop(0, NCH, chunk, 0)
    plsc.subcore_barrier()

    rows = agg_sp.at[pl.ds(sid * RPT, RPT)]

    @pl.when(cid == 0)
    def _():
        pltpu.sync_copy(rows, out0_hbm.at[pl.ds(sid * RPT, RPT)])

    @pl.when(cid == 1)
    def _():
        pltpu.sync_copy(rows, out1_hbm.at[pl.ds(sid * RPT, RPT)])


_edge_call = functools.partial(
    pl.kernel,
    out_type=(
        jax.ShapeDtypeStruct((NP, L), jnp.float32),
        jax.ShapeDtypeStruct((NP, L), jnp.float32),
    ),
    mesh=plsc.VectorSubcoreMesh(
        core_axis_name="c", subcore_axis_name="s",
        num_cores=NC, num_subcores=NS),
    scratch_types=[
        pltpu.VMEM_SHARED((NP, L), jnp.float32),
        pltpu.VMEM((C,), jnp.int32),
        pltpu.VMEM((C,), jnp.int32),
        pltpu.VMEM((C,), jnp.float32),
        pltpu.VMEM((C, L), jnp.float32),
        pltpu.VMEM((C, L), jnp.float32),
        pltpu.VMEM((2, L), jnp.float32),
        pltpu.SemaphoreType.DMA,
        pltpu.SemaphoreType.DMA,
        pltpu.SemaphoreType.DMA,
    ],
)(_edge_body)


def _dec_body(ne_ref, g0_ref, g1_ref, wu1_ref, wu2_ref, wd1a_ref, wd1b_ref,
              wd2_ref, out_ref):
    ne = ne_ref[...]
    agg = g0_ref[...] + g1_ref[...]
    lo = jnp.maximum(
        jnp.dot(ne, wu1_ref[...], preferred_element_type=jnp.float32)
        + jnp.dot(agg, wu2_ref[...], preferred_element_type=jnp.float32), 0.0)
    h = jnp.maximum(
        jnp.dot(ne, wd1a_ref[...], preferred_element_type=jnp.float32)
        + jnp.dot(lo, wd1b_ref[...], preferred_element_type=jnp.float32), 0.0)
    out_ref[...] = jnp.dot(h, wd2_ref[...], preferred_element_type=jnp.float32)


def _decode(ne, g0, g1, wu1, wu2, wd1a, wd1b, wd2p):
    row = pl.BlockSpec((RB, L), lambda i: (i, 0))
    wL = pl.BlockSpec((L, L), lambda i: (0, 0))
    return pl.pallas_call(
        _dec_body,
        grid=(NSTEPS,),
        in_specs=[row, row, row, wL, wL, wL, wL, wL],
        out_specs=row,
        out_shape=jax.ShapeDtypeStruct((N, L), jnp.float32),
    )(ne, g0, g1, wu1, wu2, wd1a, wd1b, wd2p)


def kernel(node_features, edge_features, latent_features, edge_index,
           W_node, W_edge, W_msg, W_upd, W_dec1, W_dec2):
    nf_b = jnp.broadcast_to(
        node_features.astype(jnp.float32)[:, None], (N, L))
    lat = latent_features.astype(jnp.float32)
    ne, a, b, vpn = _encode(
        nf_b, lat, W_node[0:1], W_node[1:], W_edge,
        W_msg[0:L], W_msg[L:2 * L], W_msg[2 * L:])
    src = edge_index[0].astype(jnp.int32)
    dst = edge_index[1].astype(jnp.int32)
    ef = edge_features.astype(jnp.float32)
    zer = jnp.zeros((RPT, L), jnp.float32)
    g0, g1 = _edge_call(a, b, src, dst, ef, vpn, zer)
    wd2p = jnp.pad(W_dec2, ((0, 0), (0, L - 1)))
    outp = _decode(ne, g0, g1, W_upd[:L], W_upd[L:],
                   W_dec1[:L], W_dec1[L:], wd2p)
    return outp[:, :1]


# speedup vs baseline: 3.2620x; 1.3841x over previous
import functools

import jax
import jax.numpy as jnp
from jax import lax
from jax.experimental import pallas as pl
from jax.experimental.pallas import tpu as pltpu
from jax.experimental.pallas import tpu_sc as plsc

N = 10000
E = 320000
L = 128

RB = 1000
NSTEPS = N // RB

NC = 2
NS = 16
NW = NC * NS
EPW = E // NW
C = 80
NCH = EPW // C
NP = 10240
RPT = NP // NS


def _enc_body(nfb_ref, lat_ref, wn0_ref, wn1_ref, we_ref, wm1_ref, wm2_ref,
              wm3_ref, ne_ref, a_ref, b_ref, vpn_ref):
    ne = jnp.maximum(
        nfb_ref[...] * wn0_ref[...]
        + jnp.dot(lat_ref[...], wn1_ref[...], preferred_element_type=jnp.float32),
        0.0)
    ne_ref[...] = ne
    a_ref[...] = jnp.dot(ne, wm1_ref[...], preferred_element_type=jnp.float32)
    b_ref[...] = jnp.dot(ne, wm2_ref[...], preferred_element_type=jnp.float32)
    ep = jnp.maximum(we_ref[...], 0.0)
    en = jnp.maximum(-we_ref[...], 0.0)
    vp = jnp.dot(ep, wm3_ref[...], preferred_element_type=jnp.float32)
    vn = jnp.dot(en, wm3_ref[...], preferred_element_type=jnp.float32)
    vpn_ref[...] = jnp.concatenate([vp, vn], axis=0)


def _encode(nf_b, lat, wn0, wn1, we, wm1, wm2, wm3):
    row = pl.BlockSpec((RB, L), lambda i: (i, 0))
    w1 = pl.BlockSpec((1, L), lambda i: (0, 0))
    wL = pl.BlockSpec((L, L), lambda i: (0, 0))
    return pl.pallas_call(
        _enc_body,
        grid=(NSTEPS,),
        in_specs=[row, row, w1, wL, w1, wL, wL, wL],
        out_specs=[row, row, row, pl.BlockSpec((2, L), lambda i: (0, 0))],
        out_shape=[
            jax.ShapeDtypeStruct((N, L), jnp.float32),
            jax.ShapeDtypeStruct((N, L), jnp.float32),
            jax.ShapeDtypeStruct((N, L), jnp.float32),
            jax.ShapeDtypeStruct((2, L), jnp.float32),
        ],
    )(nf_b, lat, wn0, wn1, we, wm1, wm2, wm3)


def _edge_body(a_hbm, b_hbm, src_hbm, dst_hbm, ef_hbm, vpn_hbm, zer_hbm,
               out0_hbm, out1_hbm,
               agg_sp,
               sidx0, sidx1, didx0, didx1, dsc0, dsc1, ef0, ef1,
               ar0, ar1, br0, br1, vpn_v,
               sem_i0, sem_i1, sem_a0, sem_a1, sem_b0, sem_b1,
               sem_s0, sem_s1, sem_z):
    sidx = (sidx0, sidx1)
    didx = (didx0, didx1)
    dsc = (dsc0, dsc1)
    efv = (ef0, ef1)
    ar = (ar0, ar1)
    br = (br0, br1)
    sem_i = (sem_i0, sem_i1)
    sem_a = (sem_a0, sem_a1)
    sem_b = (sem_b0, sem_b1)
    sem_s = (sem_s0, sem_s1)

    cid = lax.axis_index("c")
    sid = lax.axis_index("s")
    wid = cid * NS + sid
    ebase = wid * EPW

    pltpu.async_copy(zer_hbm, agg_sp.at[pl.ds(sid * RPT, RPT)], sem_z).wait()
    pltpu.sync_copy(vpn_hbm, vpn_v)
    plsc.subcore_barrier()

    def issue_idx(i, b):
        base = ebase + i * C
        pltpu.async_copy(src_hbm.at[pl.ds(base, C)], sidx[b], sem_i[b])
        pltpu.async_copy(dst_hbm.at[pl.ds(base, C)], didx[b], sem_i[b])
        pltpu.async_copy(ef_hbm.at[pl.ds(base, C)], efv[b], sem_i[b])

    def wait_idx(b):
        pltpu.make_async_copy(src_hbm.at[pl.ds(0, C)], sidx[b], sem_i[b]).wait()
        pltpu.make_async_copy(dst_hbm.at[pl.ds(0, C)], didx[b], sem_i[b]).wait()
        pltpu.make_async_copy(ef_hbm.at[pl.ds(0, C)], efv[b], sem_i[b]).wait()

    def issue_gathers(b):
        pltpu.async_copy(a_hbm.at[sidx[b]], ar[b], sem_a[b])
        pltpu.async_copy(b_hbm.at[didx[b]], br[b], sem_b[b])

    def wait_gathers(b):
        pltpu.make_async_copy(a_hbm.at[sidx[b]], ar[b], sem_a[b]).wait()
        pltpu.make_async_copy(b_hbm.at[didx[b]], br[b], sem_b[b]).wait()

    def wait_scatter(b):
        pltpu.make_async_copy(ar[b], agg_sp.at[dsc[b]], sem_s[b]).wait()

    def compute(b):
        arb, brb, efb = ar[b], br[b], efv[b]

        def edge16(q, c2):
            ev = efb[pl.ds(q * 16, 16)]
            for rr in range(16):
                s = ev[rr]
                sp = jnp.maximum(s, 0.0)
                sn = jnp.maximum(-s, 0.0)
                r = q * 16 + rr
                for j in range(L // 16):
                    sl = pl.ds(j * 16, 16)
                    v = (arb[r, sl] + brb[r, sl]
                         + sp * vpn_v[0, sl] + sn * vpn_v[1, sl])
                    arb[r, sl] = jnp.maximum(v, 0.0)
            return c2

        lax.fori_loop(0, C // 16, edge16, 0)

    def body(i, b):
        o = 1 - b

        @pl.when(i >= 1)
        def _():
            wait_scatter(o)

        @pl.when(i + 1 < NCH)
        def _():
            wait_idx(o)
            issue_gathers(o)

        wait_gathers(b)
        compute(b)
        for q in range(C // 16):
            sl = pl.ds(q * 16, 16)
            dsc[b][sl] = didx[b][sl]
        pltpu.async_copy(ar[b], agg_sp.at[dsc[b]], sem_s[b], add=True)

        @pl.when(i + 2 < NCH)
        def _():
            issue_idx(i + 2, b)

    issue_idx(0, 0)
    issue_idx(1, 1)
    wait_idx(0)
    issue_gathers(0)

    def pair(t, carry):
        body(2 * t, 0)
        body(2 * t + 1, 1)
        return carry

    lax.fori_loop(0, NCH // 2, pair, 0)
    body(jnp.int32(NCH - 1), 0)
    wait_scatter(0)
    plsc.subcore_barrier()

    rows = agg_sp.at[pl.ds(sid * RPT, RPT)]

    @pl.when(cid == 0)
    def _():
        pltpu.sync_copy(rows, out0_hbm.at[pl.ds(sid * RPT, RPT)])

    @pl.when(cid == 1)
    def _():
        pltpu.sync_copy(rows, out1_hbm.at[pl.ds(sid * RPT, RPT)])


_edge_call = functools.partial(
    pl.kernel,
    out_type=(
        jax.ShapeDtypeStruct((NP, L), jnp.float32),
        jax.ShapeDtypeStruct((NP, L), jnp.float32),
    ),
    mesh=plsc.VectorSubcoreMesh(
        core_axis_name="c", subcore_axis_name="s",
        num_cores=NC, num_subcores=NS),
    scratch_types=(
        [pltpu.VMEM_SHARED((NP, L), jnp.float32)]
        + [pltpu.VMEM((C,), jnp.int32)] * 6
        + [pltpu.VMEM((C,), jnp.float32)] * 2
        + [pltpu.VMEM((C, L), jnp.float32)] * 4
        + [pltpu.VMEM((2, L), jnp.float32)]
        + [pltpu.SemaphoreType.DMA] * 9
    ),
)(_edge_body)


def _dec_body(ne_ref, g0_ref, g1_ref, wu1_ref, wu2_ref, wd1a_ref, wd1b_ref,
              wd2_ref, out_ref):
    ne = ne_ref[...]
    agg = g0_ref[...] + g1_ref[...]
    lo = jnp.maximum(
        jnp.dot(ne, wu1_ref[...], preferred_element_type=jnp.float32)
        + jnp.dot(agg, wu2_ref[...], preferred_element_type=jnp.float32), 0.0)
    h = jnp.maximum(
        jnp.dot(ne, wd1a_ref[...], preferred_element_type=jnp.float32)
        + jnp.dot(lo, wd1b_ref[...], preferred_element_type=jnp.float32), 0.0)
    out_ref[...] = jnp.dot(h, wd2_ref[...], preferred_element_type=jnp.float32)


def _decode(ne, g0, g1, wu1, wu2, wd1a, wd1b, wd2p):
    row = pl.BlockSpec((RB, L), lambda i: (i, 0))
    wL = pl.BlockSpec((L, L), lambda i: (0, 0))
    return pl.pallas_call(
        _dec_body,
        grid=(NSTEPS,),
        in_specs=[row, row, row, wL, wL, wL, wL, wL],
        out_specs=row,
        out_shape=jax.ShapeDtypeStruct((N, L), jnp.float32),
    )(ne, g0, g1, wu1, wu2, wd1a, wd1b, wd2p)


def kernel(node_features, edge_features, latent_features, edge_index,
           W_node, W_edge, W_msg, W_upd, W_dec1, W_dec2):
    nf_b = jnp.broadcast_to(
        node_features.astype(jnp.float32)[:, None], (N, L))
    lat = latent_features.astype(jnp.float32)
    ne, a, b, vpn = _encode(
        nf_b, lat, W_node[0:1], W_node[1:], W_edge,
        W_msg[0:L], W_msg[L:2 * L], W_msg[2 * L:])
    src = edge_index[0].astype(jnp.int32)
    dst = edge_index[1].astype(jnp.int32)
    ef = edge_features.astype(jnp.float32)
    zer = jnp.zeros((RPT, L), jnp.float32)
    g0, g1 = _edge_call(a, b, src, dst, ef, vpn, zer)
    wd2p = jnp.pad(W_dec2, ((0, 0), (0, L - 1)))
    outp = _decode(ne, g0, g1, W_upd[:L], W_upd[L:],
                   W_dec1[:L], W_dec1[L:], wd2p)
    return outp[:, :1]


# speedup vs baseline: 3.2834x; 1.0065x over previous
import functools

import jax
import jax.numpy as jnp
from jax import lax
from jax.experimental import pallas as pl
from jax.experimental.pallas import tpu as pltpu
from jax.experimental.pallas import tpu_sc as plsc

N = 10000
E = 320000
L = 128

RB = 1000
NSTEPS = N // RB

NC = 2
NS = 16
NW = NC * NS
EPW = E // NW
C = 80
NCH = EPW // C
NP = 10240
RPT = NP // NS


def _enc_body(nfb_ref, lat_ref, wn0_ref, wn1_ref, we_ref, wm1_ref, wm2_ref,
              wm3_ref, ne_ref, a_ref, b_ref, vpn_ref):
    ne = jnp.maximum(
        nfb_ref[...] * wn0_ref[...]
        + jnp.dot(lat_ref[...], wn1_ref[...], preferred_element_type=jnp.float32),
        0.0)
    ne_ref[...] = ne
    a_ref[...] = jnp.dot(ne, wm1_ref[...], preferred_element_type=jnp.float32)
    b_ref[...] = jnp.dot(ne, wm2_ref[...], preferred_element_type=jnp.float32)
    ep = jnp.maximum(we_ref[...], 0.0)
    en = jnp.maximum(-we_ref[...], 0.0)
    vp = jnp.dot(ep, wm3_ref[...], preferred_element_type=jnp.float32)
    vn = jnp.dot(en, wm3_ref[...], preferred_element_type=jnp.float32)
    vpn_ref[...] = jnp.concatenate([vp, vn], axis=0)


def _encode(nf_b, lat, wn0, wn1, we, wm1, wm2, wm3):
    row = pl.BlockSpec((RB, L), lambda i: (i, 0))
    w1 = pl.BlockSpec((1, L), lambda i: (0, 0))
    wL = pl.BlockSpec((L, L), lambda i: (0, 0))
    return pl.pallas_call(
        _enc_body,
        grid=(NSTEPS,),
        in_specs=[row, row, w1, wL, w1, wL, wL, wL],
        out_specs=[row, row, row, pl.BlockSpec((2, L), lambda i: (0, 0))],
        out_shape=[
            jax.ShapeDtypeStruct((N, L), jnp.float32),
            jax.ShapeDtypeStruct((N, L), jnp.float32),
            jax.ShapeDtypeStruct((N, L), jnp.float32),
            jax.ShapeDtypeStruct((2, L), jnp.float32),
        ],
    )(nf_b, lat, wn0, wn1, we, wm1, wm2, wm3)


def _edge_body(a_hbm, b_hbm, src_hbm, dst_hbm, ef_hbm, vpn_hbm, zer_hbm,
               out0_hbm, out1_hbm,
               agg_sp,
               sidx0, sidx1, didx0, didx1, dsc0, dsc1, ef0, ef1,
               ar0, ar1, br0, br1, vpn_v,
               sem_i0, sem_i1, sem_a0, sem_a1, sem_b0, sem_b1,
               sem_s0, sem_s1, sem_z):
    sidx = (sidx0, sidx1)
    didx = (didx0, didx1)
    dsc = (dsc0, dsc1)
    efv = (ef0, ef1)
    ar = (ar0, ar1)
    br = (br0, br1)
    sem_i = (sem_i0, sem_i1)
    sem_a = (sem_a0, sem_a1)
    sem_b = (sem_b0, sem_b1)
    sem_s = (sem_s0, sem_s1)

    cid = lax.axis_index("c")
    sid = lax.axis_index("s")
    wid = cid * NS + sid
    ebase = wid * EPW

    pltpu.async_copy(zer_hbm, agg_sp.at[pl.ds(sid * RPT, RPT)], sem_z).wait()
    pltpu.sync_copy(vpn_hbm, vpn_v)
    plsc.subcore_barrier()

    def issue_idx(i, b):
        base = ebase + i * C
        pltpu.async_copy(src_hbm.at[pl.ds(base, C)], sidx[b], sem_i[b])
        pltpu.async_copy(dst_hbm.at[pl.ds(base, C)], didx[b], sem_i[b])
        pltpu.async_copy(ef_hbm.at[pl.ds(base, C)], efv[b], sem_i[b])

    def wait_idx(b):
        pltpu.make_async_copy(src_hbm.at[pl.ds(0, C)], sidx[b], sem_i[b]).wait()
        pltpu.make_async_copy(dst_hbm.at[pl.ds(0, C)], didx[b], sem_i[b]).wait()
        pltpu.make_async_copy(ef_hbm.at[pl.ds(0, C)], efv[b], sem_i[b]).wait()

    def issue_gathers(b):
        pltpu.async_copy(a_hbm.at[sidx[b]], ar[b], sem_a[b])
        pltpu.async_copy(b_hbm.at[didx[b]], br[b], sem_b[b])

    def wait_gathers(b):
        pltpu.make_async_copy(a_hbm.at[sidx[b]], ar[b], sem_a[b]).wait()
        pltpu.make_async_copy(b_hbm.at[didx[b]], br[b], sem_b[b]).wait()

    def wait_scatter(b):
        pltpu.make_async_copy(ar[b], agg_sp.at[dsc[b]], sem_s[b]).wait()

    def compute(b):
        arb, brb, efb = ar[b], br[b], efv[b]

        def edge16(q, c2):
            ev = efb[pl.ds(q * 16, 16)]
            spv = jnp.maximum(ev, 0.0)
            snv = jnp.maximum(-ev, 0.0)
            for rr in range(16):
                lane = jnp.full((16,), rr, jnp.int32)
                sp = spv.at[lane].get(mode="promise_in_bounds")
                sn = snv.at[lane].get(mode="promise_in_bounds")
                r = q * 16 + rr
                for j in range(L // 16):
                    sl = pl.ds(j * 16, 16)
                    v = (arb[r, sl] + brb[r, sl]
                         + sp * vpn_v[0, sl] + sn * vpn_v[1, sl])
                    arb[r, sl] = jnp.maximum(v, 0.0)
            return c2

        lax.fori_loop(0, C // 16, edge16, 0)

    def body(i, b):
        o = 1 - b

        @pl.when(i >= 1)
        def _():
            wait_scatter(o)

        @pl.when(i + 1 < NCH)
        def _():
            wait_idx(o)
            issue_gathers(o)

        wait_gathers(b)
        compute(b)
        for q in range(C // 16):
            sl = pl.ds(q * 16, 16)
            dsc[b][sl] = didx[b][sl]
        pltpu.async_copy(ar[b], agg_sp.at[dsc[b]], sem_s[b], add=True)

        @pl.when(i + 2 < NCH)
        def _():
            issue_idx(i + 2, b)

    issue_idx(0, 0)
    issue_idx(1, 1)
    wait_idx(0)
    issue_gathers(0)

    def pair(t, carry):
        body(2 * t, 0)
        body(2 * t + 1, 1)
        return carry

    lax.fori_loop(0, NCH // 2, pair, 0)
    body(jnp.int32(NCH - 1), 0)
    wait_scatter(0)
    plsc.subcore_barrier()

    rows = agg_sp.at[pl.ds(sid * RPT, RPT)]

    @pl.when(cid == 0)
    def _():
        pltpu.sync_copy(rows, out0_hbm.at[pl.ds(sid * RPT, RPT)])

    @pl.when(cid == 1)
    def _():
        pltpu.sync_copy(rows, out1_hbm.at[pl.ds(sid * RPT, RPT)])


_edge_call = functools.partial(
    pl.kernel,
    out_type=(
        jax.ShapeDtypeStruct((NP, L), jnp.float32),
        jax.ShapeDtypeStruct((NP, L), jnp.float32),
    ),
    mesh=plsc.VectorSubcoreMesh(
        core_axis_name="c", subcore_axis_name="s",
        num_cores=NC, num_subcores=NS),
    scratch_types=(
        [pltpu.VMEM_SHARED((NP, L), jnp.float32)]
        + [pltpu.VMEM((C,), jnp.int32)] * 6
        + [pltpu.VMEM((C,), jnp.float32)] * 2
        + [pltpu.VMEM((C, L), jnp.float32)] * 4
        + [pltpu.VMEM((2, L), jnp.float32)]
        + [pltpu.SemaphoreType.DMA] * 9
    ),
)(_edge_body)


def _dec_body(ne_ref, g0_ref, g1_ref, wu1_ref, wu2_ref, wd1a_ref, wd1b_ref,
              wd2_ref, out_ref):
    ne = ne_ref[...]
    agg = g0_ref[...] + g1_ref[...]
    lo = jnp.maximum(
        jnp.dot(ne, wu1_ref[...], preferred_element_type=jnp.float32)
        + jnp.dot(agg, wu2_ref[...], preferred_element_type=jnp.float32), 0.0)
    h = jnp.maximum(
        jnp.dot(ne, wd1a_ref[...], preferred_element_type=jnp.float32)
        + jnp.dot(lo, wd1b_ref[...], preferred_element_type=jnp.float32), 0.0)
    out_ref[...] = jnp.dot(h, wd2_ref[...], preferred_element_type=jnp.float32)


def _decode(ne, g0, g1, wu1, wu2, wd1a, wd1b, wd2p):
    row = pl.BlockSpec((RB, L), lambda i: (i, 0))
    wL = pl.BlockSpec((L, L), lambda i: (0, 0))
    return pl.pallas_call(
        _dec_body,
        grid=(NSTEPS,),
        in_specs=[row, row, row, wL, wL, wL, wL, wL],
        out_specs=row,
        out_shape=jax.ShapeDtypeStruct((N, L), jnp.float32),
    )(ne, g0, g1, wu1, wu2, wd1a, wd1b, wd2p)


def kernel(node_features, edge_features, latent_features, edge_index,
           W_node, W_edge, W_msg, W_upd, W_dec1, W_dec2):
    nf_b = jnp.broadcast_to(
        node_features.astype(jnp.float32)[:, None], (N, L))
    lat = latent_features.astype(jnp.float32)
    ne, a, b, vpn = _encode(
        nf_b, lat, W_node[0:1], W_node[1:], W_edge,
        W_msg[0:L], W_msg[L:2 * L], W_msg[2 * L:])
    src = edge_index[0].astype(jnp.int32)
    dst = edge_index[1].astype(jnp.int32)
    ef = edge_features.astype(jnp.float32)
    zer = jnp.zeros((RPT, L), jnp.float32)
    g0, g1 = _edge_call(a, b, src, dst, ef, vpn, zer)
    wd2p = jnp.pad(W_dec2, ((0, 0), (0, L - 1)))
    outp = _decode(ne, g0, g1, W_upd[:L], W_upd[L:],
                   W_dec1[:L], W_dec1[L:], wd2p)
    return outp[:, :1]
